# baseline (speedup 1.0000x reference)
import functools

import jax
import jax.numpy as jnp
from jax import lax
from jax.experimental import pallas as pl
from jax.experimental.pallas import tpu as pltpu
from jax.experimental.pallas import tpu_sc as plsc

N_ENV = 64
BUF = 4096
N_OBS = 128
N_ACT = 32
BATCH = 256
OUT_D = N_OBS + N_ACT + N_OBS + 3
DROW_PAD = 384
L = 16
HB = 128
NH = HB // L


def _build_kernel(num_cores, num_subcores):
    n_workers = num_cores * num_subcores
    epw = N_ENV // n_workers
    mesh = plsc.VectorSubcoreMesh(core_axis_name="c", subcore_axis_name="s")
    f32 = jnp.float32
    i32 = jnp.int32

    @functools.partial(
        pl.kernel,
        out_type=[
            jax.ShapeDtypeStruct((N_ENV * BATCH, N_OBS), f32),
            jax.ShapeDtypeStruct((N_ENV * BATCH, N_ACT), f32),
            jax.ShapeDtypeStruct((N_ENV * BATCH, N_OBS), f32),
            jax.ShapeDtypeStruct((N_ENV * BATCH,), f32),
            jax.ShapeDtypeStruct((N_ENV * BATCH,), f32),
            jax.ShapeDtypeStruct((N_ENV * BATCH,), f32),
        ],
        mesh=mesh,
        compiler_params=pltpu.CompilerParams(needs_layout_passes=False),
        scratch_types=[
            pltpu.VMEM((epw,), i32),
            pltpu.VMEM((1,), i32),
            pltpu.VMEM((epw, BATCH), i32),
            pltpu.VMEM((1, BUF), f32),
            pltpu.VMEM((1, BUF), i32),
            pltpu.VMEM((1, BUF), i32),
            pltpu.VMEM((epw, DROW_PAD), f32),
            pltpu.VMEM((HB,), i32),
            pltpu.VMEM((HB,), i32),
            pltpu.VMEM((HB, N_OBS), f32),
            pltpu.VMEM((HB, 128), f32),
            pltpu.VMEM((HB, N_ACT), f32),
            pltpu.VMEM((HB, N_OBS), f32),
            pltpu.VMEM((HB,), f32),
            pltpu.VMEM((HB,), f32),
            pltpu.VMEM((HB,), f32),
            pltpu.VMEM((L,), i32),
            pltpu.SemaphoreType.DMA,
            pltpu.SemaphoreType.DMA,
        ],
    )
    def k(obs_hbm, act_hbm, nobs_hbm, rew_hbm, dn_hbm, tr_hbm, data_hbm,
          tvec_hbm, sidx_hbm,
          o_obs, o_act, o_nobs, o_rew, o_dn, o_tr,
          eidx, eidx1, sidx2, rew1, dn1, tr1, data2, gidx, pgidx,
          obs_stage, act_wide, act_stage, nobs_stage,
          rew_o, dn_o, tr_o, tv, sem, sem2):
        wid = lax.axis_index("s") * num_cores + lax.axis_index("c")
        lane = lax.iota(i32, L)
        zero = jnp.full((L,), 0, i32)
        e0 = wid * epw
        plsc.store_scatter(eidx, [lane], e0 + lane, mask=lane < epw)
        stage = [
            pltpu.async_copy(sidx_hbm.at[eidx], sidx2, sem),
            pltpu.async_copy(data_hbm.at[eidx], data2, sem),
        ]
        pltpu.sync_copy(tvec_hbm, tv)
        tvec = tv[...]
        for c in stage:
            c.wait()

        for j in range(epw):
            e = e0 + j
            ebase = e * BUF
            plsc.store_scatter(eidx1, [lane], (e0 + j) + zero, mask=lane < 1)
            scopies = [
                pltpu.async_copy(rew_hbm.at[eidx1], rew1, sem2),
                pltpu.async_copy(dn_hbm.at[eidx1], dn1, sem2),
                pltpu.async_copy(tr_hbm.at[eidx1], tr1, sem2),
            ]
            jv = jnp.full((L,), j, i32)

            for h in range(2):
                for kk in range(NH):
                    s = pl.ds(kk * L, L)
                    v = sidx2[j, pl.ds(h * HB + kk * L, L)] + ebase
                    gidx[s] = v
                    pgidx[s] = lax.shift_right_logical(v, 2)
                copies = [
                    pltpu.async_copy(obs_hbm.at[gidx], obs_stage, sem),
                    pltpu.async_copy(act_hbm.at[pgidx], act_wide, sem),
                    pltpu.async_copy(nobs_hbm.at[gidx], nobs_stage, sem),
                ]
                if h == 0:
                    for c in scopies:
                        c.wait()
                for kk in range(NH):
                    s = pl.ds(kk * L, L)
                    ii = sidx2[j, pl.ds(h * HB + kk * L, L)]
                    rew_o[s] = plsc.load_gather(rew1, [zero, ii])
                    dn_o[s] = plsc.load_gather(dn1, [zero, ii]).astype(f32)
                    tr_o[s] = plsc.load_gather(tr1, [zero, ii]).astype(f32)
                for c in copies:
                    c.wait()

                def act_body(kk, _):
                    rows16 = kk * L + lane
                    ii = sidx2[j, pl.ds(h * HB + kk * L, L)]
                    off = (ii & 3) * N_ACT
                    for jj in range(N_ACT):
                        vals = plsc.load_gather(act_wide, [rows16, off + jj])
                        plsc.store_scatter(
                            act_stage, [rows16, jnp.full((L,), jj, i32)],
                            vals)
                    return 0

                lax.fori_loop(0, NH, act_body, 0)

                def patch_chunk(kk, _):
                    ii = sidx2[j, pl.ds(h * HB + kk * L, L)]
                    m = (ii == tvec).astype(i32)
                    nm = jnp.sum(m)

                    @pl.when(nm > 0)
                    def _():
                        def per_lane(l, _):
                            ml = jnp.sum(jnp.where(lane == l, m, 0))

                            @pl.when(ml > 0)
                            def _():
                                b = jnp.full((L,), kk * L + l, i32)

                                def cp(base, n, ref):
                                    def body(c, _):
                                        cols = c * L + lane
                                        plsc.store_scatter(
                                            ref, [b, cols],
                                            plsc.load_gather(
                                                data2, [jv, base + cols]))
                                        return 0
                                    lax.fori_loop(0, n // L, body, 0)

                                cp(0, N_OBS, obs_stage)
                                cp(N_OBS, N_ACT, act_stage)
                                cp(N_OBS + N_ACT, N_OBS, nobs_stage)
                                c0 = N_OBS + N_ACT + N_OBS
                                m0 = lane == 0
                                for t, ref in enumerate((rew_o, dn_o, tr_o)):
                                    plsc.store_scatter(
                                        ref, [b],
                                        plsc.load_gather(
                                            data2,
                                            [jv, jnp.full((L,), c0 + t,
                                                          i32)]),
                                        mask=m0)
                            return 0

                        lax.fori_loop(0, L, per_lane, 0)
                    return 0

                lax.fori_loop(0, NH, patch_chunk, 0)

                orow = pl.ds(e * BATCH + h * HB, HB)
                pltpu.sync_copy(obs_stage, o_obs.at[orow])
                pltpu.sync_copy(act_stage, o_act.at[orow])
                pltpu.sync_copy(nobs_stage, o_nobs.at[orow])
                pltpu.sync_copy(rew_o, o_rew.at[orow])
                pltpu.sync_copy(dn_o, o_dn.at[orow])
                pltpu.sync_copy(tr_o, o_tr.at[orow])

    return k


def kernel(observations, actions, rewards, dones, truncations,
           next_observations, obs_data, act_data, next_obs_data, rewards_data,
           dones_data, truncations_data, ptr, sample_idx):
    info = plsc.get_sparse_core_info()
    k = _build_kernel(info.num_cores, info.num_subcores)
    t = jnp.asarray(ptr, jnp.int32) % BUF
    tvec = jnp.full((L,), t, jnp.int32)
    data_comb = jnp.concatenate([
        obs_data, act_data, next_obs_data,
        rewards_data[:, None],
        dones_data[:, None].astype(jnp.float32),
        truncations_data[:, None].astype(jnp.float32),
        jnp.zeros((N_ENV, DROW_PAD - OUT_D), jnp.float32),
    ], axis=1)
    s_obs, s_act, s_nobs, s_rew, s_dn, s_tr = k(
        observations.reshape(N_ENV * BUF, N_OBS),
        actions.reshape(N_ENV * BUF // 4, 128),
        next_observations.reshape(N_ENV * BUF, N_OBS),
        rewards, dones, truncations, data_comb, tvec,
        sample_idx.astype(jnp.int32))
    return jnp.concatenate(
        [s_obs, s_act, s_nobs, s_rew[:, None], s_dn[:, None], s_tr[:, None]],
        axis=-1)
